# baseline (speedup 1.0000x reference)
import functools

import jax
import jax.numpy as jnp
from jax import lax
from jax.experimental import pallas as pl
from jax.experimental.pallas import tpu as pltpu
from jax.experimental.pallas import tpu_sc as plsc

_B, _L, _F, _R = 32, 64, 64, 256
_NB_SC = 16
_B_TC = _B - _NB_SC
_WPB = 32 // _NB_SC
_FS = _F // _WPB
_G = 4

_mesh = plsc.VectorSubcoreMesh(core_axis_name="c", subcore_axis_name="s")


_LH = _L // 2
_FH = _F // 2


@functools.partial(
    pl.kernel,
    out_type=jax.ShapeDtypeStruct((_NB_SC, 2, 2, _F, _R), jnp.float32),
    mesh=_mesh,
    scratch_types=[
        pltpu.VMEM((2, _FH, _R), jnp.float32),
        pltpu.VMEM((_F, _R), jnp.float32),
        pltpu.VMEM((_F, _R), jnp.float32),
        pltpu.VMEM((2, _L, 16), jnp.float32),
        pltpu.SemaphoreType.DMA,
        pltpu.SemaphoreType.DMA,
    ],
)
def _sc_reduce(x_hbm, wx_hbm, args_hbm, xbuf, acc1, acc2, wv, sem0, sem1):
    wid = lax.axis_index("s") * 2 + lax.axis_index("c")
    bi = wid // 2
    b = _B_TC + bi
    h = wid % 2
    l0 = h * _LH
    nch = 2 * _LH

    pltpu.sync_copy(wx_hbm.at[b], wv)

    zv = jnp.zeros((16,), jnp.float32)

    def zero_row(f, _):
        for c in range(_R // 16):
            acc1[f, pl.ds(c * 16, 16)] = zv
            acc2[f, pl.ds(c * 16, 16)] = zv
        return 0

    lax.fori_loop(0, _F, zero_row, 0)

    def chunk_src(k):
        return x_hbm.at[b, l0 + k // 2, pl.ds((k % 2) * _FH, _FH)]

    pltpu.async_copy(chunk_src(0), xbuf.at[0], sem0)
    pltpu.async_copy(chunk_src(1), xbuf.at[1], sem1)

    def accum(k, p, sem):
        pltpu.make_async_copy(chunk_src(k), xbuf.at[p], sem).wait()
        l = l0 + k // 2
        fb = (k % 2) * _FH
        w1 = wv[0, l, pl.ds(0, 16)]
        w2 = wv[1, l, pl.ds(0, 16)]

        def frow(f, _):
            for c in range(_R // 16):
                ds = pl.ds(c * 16, 16)
                xv = xbuf[p, f, ds]
                acc1[fb + f, ds] = acc1[fb + f, ds] + w1 * xv
                acc2[fb + f, ds] = acc2[fb + f, ds] + w2 * xv
            return 0

        lax.fori_loop(0, _FH, frow, 0)

        @pl.when(k < nch - 2)
        def _():
            pltpu.async_copy(chunk_src(k + 2), xbuf.at[p], sem)

    def step(k2, _):
        accum(2 * k2, 0, sem0)
        accum(2 * k2 + 1, 1, sem1)
        return 0

    lax.fori_loop(0, nch // 2, step, 0)

    pltpu.sync_copy(acc1, args_hbm.at[bi, h, 0])
    pltpu.sync_copy(acc2, args_hbm.at[bi, h, 1])


def _tc_body(ws_ref, wv_ref, x_ref, m_ref, rf_ref, rr_ref,
             car_ref, cdr_ref, cons_ref, max_ref):
    b = pl.program_id(0)

    def step(l, accs):
        a1, a2 = accs
        xl = x_ref[0, l]
        return (a1 + ws_ref[b, 0, l] * xl, a2 + ws_ref[b, 1, l] * xl)

    z = jnp.zeros((_F, _R), jnp.float32)
    a1, a2 = lax.fori_loop(0, _L, step, (z, z))
    car_ref[0] = jnp.dot(a1, m_ref[0], preferred_element_type=jnp.float32)
    cdr_ref[0] = jnp.dot(a2, m_ref[1], preferred_element_type=jnp.float32)
    cons_ref[0] = (
        jnp.dot(a1, m_ref[2], preferred_element_type=jnp.float32)
        + jnp.dot(a2, m_ref[3], preferred_element_type=jnp.float32)
        + rf_ref[0] * rr_ref[...])

    @pl.when(b == 0)
    def _():
        max_ref[...] = jnp.max(wv_ref[...], axis=-1)


def _tc_tail(args_ref, m_ref, rf_ref, rr_ref, car_ref, cdr_ref, cons_ref):
    a1 = args_ref[0, 0, 0] + args_ref[0, 1, 0]
    a2 = args_ref[0, 0, 1] + args_ref[0, 1, 1]
    car_ref[0] = jnp.dot(a1, m_ref[0], preferred_element_type=jnp.float32)
    cdr_ref[0] = jnp.dot(a2, m_ref[1], preferred_element_type=jnp.float32)
    cons_ref[0] = (
        jnp.dot(a1, m_ref[2], preferred_element_type=jnp.float32)
        + jnp.dot(a2, m_ref[3], preferred_element_type=jnp.float32)
        + rf_ref[0] * rr_ref[...])


def kernel(x, arg1_weight, arg2_weight, root_filler, D_l, D_r, E_l, E_r, root_role):
    B, L, F, R = _B, _L, _F, _R
    W = jnp.stack([arg1_weight, arg2_weight], axis=1)
    mats = jnp.stack([D_l.T, D_r.T, E_l.T, E_r.T], axis=0)
    rf = root_filler.reshape(B, F, 1)
    rr = root_role.reshape(1, R)

    Wx = jnp.broadcast_to(W[..., None], (B, 2, L, 16))
    args_sc = _sc_reduce(x, Wx)

    car_tc, cdr_tc, cons_tc, maxes = pl.pallas_call(
        _tc_body,
        grid=(_B_TC,),
        in_specs=[
            pl.BlockSpec(memory_space=pltpu.SMEM),
            pl.BlockSpec((B, 2, L), lambda b: (0, 0, 0)),
            pl.BlockSpec((1, L, F, R), lambda b: (b, 0, 0, 0)),
            pl.BlockSpec((4, R, R), lambda b: (0, 0, 0)),
            pl.BlockSpec((1, F, 1), lambda b: (b, 0, 0)),
            pl.BlockSpec((1, R), lambda b: (0, 0)),
        ],
        out_specs=[
            pl.BlockSpec((1, F, R), lambda b: (b, 0, 0)),
            pl.BlockSpec((1, F, R), lambda b: (b, 0, 0)),
            pl.BlockSpec((1, F, R), lambda b: (b, 0, 0)),
            pl.BlockSpec((B, 2), lambda b: (0, 0)),
        ],
        out_shape=[
            jax.ShapeDtypeStruct((_B_TC, F, R), jnp.float32),
            jax.ShapeDtypeStruct((_B_TC, F, R), jnp.float32),
            jax.ShapeDtypeStruct((_B_TC, F, R), jnp.float32),
            jax.ShapeDtypeStruct((B, 2), jnp.float32),
        ],
    )(W, W, x, mats, rf, rr)

    car_sc, cdr_sc, cons_sc = pl.pallas_call(
        _tc_tail,
        grid=(_NB_SC,),
        in_specs=[
            pl.BlockSpec((1, 2, 2, F, R), lambda b: (b, 0, 0, 0, 0)),
            pl.BlockSpec((4, R, R), lambda b: (0, 0, 0)),
            pl.BlockSpec((1, F, 1), lambda b: (_B_TC + b, 0, 0)),
            pl.BlockSpec((1, R), lambda b: (0, 0)),
        ],
        out_specs=[
            pl.BlockSpec((1, F, R), lambda b: (b, 0, 0)),
            pl.BlockSpec((1, F, R), lambda b: (b, 0, 0)),
            pl.BlockSpec((1, F, R), lambda b: (b, 0, 0)),
        ],
        out_shape=[
            jax.ShapeDtypeStruct((_NB_SC, F, R), jnp.float32),
            jax.ShapeDtypeStruct((_NB_SC, F, R), jnp.float32),
            jax.ShapeDtypeStruct((_NB_SC, F, R), jnp.float32),
        ],
    )(args_sc, mats, rf, rr)

    car = jnp.concatenate([car_tc, car_sc], axis=0)
    cdr = jnp.concatenate([cdr_tc, cdr_sc], axis=0)
    cons = jnp.concatenate([cons_tc, cons_sc], axis=0)
    return (car, cdr, cons, maxes[:, 0], maxes[:, 1])


# speedup vs baseline: 1.5414x; 1.5414x over previous
import jax
import jax.numpy as jnp
from jax import lax
from jax.experimental import pallas as pl
from jax.experimental.pallas import tpu as pltpu

_B, _L, _F, _R = 32, 64, 64, 256
_BB = 2


def _body(ws_ref, wv_ref, x_ref, m_ref, rf_ref, rr_ref,
          car_ref, cdr_ref, cons_ref, max_ref):
    g = pl.program_id(0)

    for i in range(_BB):
        b = g * _BB + i

        def step(l, accs):
            a1, a2 = accs
            xl = x_ref[i, l]
            return (a1 + ws_ref[b, 0, l] * xl, a2 + ws_ref[b, 1, l] * xl)

        z = jnp.zeros((_F, _R), jnp.float32)
        a1, a2 = lax.fori_loop(0, _L, step, (z, z))
        car_ref[i] = jnp.dot(a1, m_ref[0], preferred_element_type=jnp.float32)
        cdr_ref[i] = jnp.dot(a2, m_ref[1], preferred_element_type=jnp.float32)
        cons_ref[i] = (
            jnp.dot(a1, m_ref[2], preferred_element_type=jnp.float32)
            + jnp.dot(a2, m_ref[3], preferred_element_type=jnp.float32)
            + rf_ref[i] * rr_ref[...])

    @pl.when(g == 0)
    def _():
        max_ref[...] = jnp.max(wv_ref[...], axis=-1)


def kernel(x, arg1_weight, arg2_weight, root_filler, D_l, D_r, E_l, E_r, root_role):
    B, L, F, R = _B, _L, _F, _R
    W = jnp.stack([arg1_weight, arg2_weight], axis=1)
    mats = jnp.stack([D_l.T, D_r.T, E_l.T, E_r.T], axis=0)
    rf = root_filler.reshape(B, F, 1)
    rr = root_role.reshape(1, R)
    car, cdr, cons, maxes = pl.pallas_call(
        _body,
        grid=(B // _BB,),
        in_specs=[
            pl.BlockSpec(memory_space=pltpu.SMEM),
            pl.BlockSpec((B, 2, L), lambda g: (0, 0, 0)),
            pl.BlockSpec((_BB, L, F, R), lambda g: (g, 0, 0, 0)),
            pl.BlockSpec((4, R, R), lambda g: (0, 0, 0)),
            pl.BlockSpec((_BB, F, 1), lambda g: (g, 0, 0)),
            pl.BlockSpec((1, R), lambda g: (0, 0)),
        ],
        out_specs=[
            pl.BlockSpec((_BB, F, R), lambda g: (g, 0, 0)),
            pl.BlockSpec((_BB, F, R), lambda g: (g, 0, 0)),
            pl.BlockSpec((_BB, F, R), lambda g: (g, 0, 0)),
            pl.BlockSpec((B, 2), lambda g: (0, 0)),
        ],
        out_shape=[
            jax.ShapeDtypeStruct((B, F, R), jnp.float32),
            jax.ShapeDtypeStruct((B, F, R), jnp.float32),
            jax.ShapeDtypeStruct((B, F, R), jnp.float32),
            jax.ShapeDtypeStruct((B, 2), jnp.float32),
        ],
    )(W, W, x, mats, rf, rr)
    return (car, cdr, cons, maxes[:, 0], maxes[:, 1])
